# baseline (speedup 1.0000x reference)
import jax
import jax.numpy as jnp
from jax.experimental import pallas as pl
from jax.experimental.pallas import tpu as pltpu
from jax.experimental.pallas import tpu_sc as plsc

P_ROWS = 4096
N_ROWS = 200
D_MODEL = 256
D8 = D_MODEL // 8
D_REL = 64
NUM_IDX = P_ROWS * N_ROWS
GATHER_W = 128

_OFF_M, _OFF_W, _OFF_H, _OFF_MIN, _OFF_S = 0, 12, 17, 41, 101
_COMB = 105
_COMB_PAD = 128


def _relidx_body(pts_ref, nts_t_ref, out_ref):
    mb = pts_ref[:, 2:3] * 60 + pts_ref[:, 3:4]
    ma = nts_t_ref[2:3, :] * 60 + nts_t_ref[3:4, :]
    out_ref[...] = jnp.clip(mb - ma, -500, 500) + 500


def _session_col(hour, minute):
    t = hour * 60 + minute
    return jnp.where(t < 4 * 60, 0,
           jnp.where(t < 9 * 60 + 30, 1,
           jnp.where(t < 16 * 60, 2,
           jnp.where(t < 20 * 60, 3, 0))))


def _embed_body(pts_ref, nts_ref, bdiag_ref, w_ref, b_ref, scale_ref,
                pout_ref, nout_ref):
    bdiag = bdiag_ref[...]
    w = w_ref[...]
    bias = b_ref[...]

    def emb(ts, nrows, scale_val):
        cm = ts[:, 0:1] - 1 + _OFF_M
        cw = ts[:, 1:2] + _OFF_W
        ch = ts[:, 2:3] + _OFF_H
        cmin = ts[:, 3:4] + _OFF_MIN
        cs = _session_col(ts[:, 2:3], ts[:, 3:4]) + _OFF_S
        col = jax.lax.broadcasted_iota(jnp.int32, (nrows, _COMB_PAD), 1)
        h = ((col == cm).astype(jnp.float32)
             + (col == cw).astype(jnp.float32)
             + (col == ch).astype(jnp.float32)
             + (col == cmin).astype(jnp.float32)
             + (col == cs).astype(jnp.float32))
        feats = jnp.dot(h, bdiag, preferred_element_type=jnp.float32)
        out = jnp.dot(feats, w, preferred_element_type=jnp.float32)
        return (out + bias) * scale_val

    pout_ref[...] = emb(pts_ref[...], P_ROWS, scale_ref[1])
    nout_ref[...] = emb(nts_ref[...], N_ROWS, scale_ref[0])


_NW = 32
_PPW = P_ROWS // _NW
_CP = 2
_CROWS = _CP * N_ROWS
_NBUF = 2
_NCH = _PPW // _CP
_TROWS = 1001
_L = 16
_ROWW = N_ROWS * D_REL


def _sc_gather(table, idx_flat):
    mesh = plsc.VectorSubcoreMesh(core_axis_name="c", subcore_axis_name="s")

    @pl.kernel(
        out_type=jax.ShapeDtypeStruct((P_ROWS, N_ROWS, D_REL), jnp.float32),
        mesh=mesh,
        scratch_types=[
            pltpu.VMEM((_TROWS, D_REL), jnp.float32),
            pltpu.VMEM((_NBUF, _CROWS), jnp.int32),
            pltpu.VMEM((_NBUF, _CROWS, D_REL), jnp.float32),
            pltpu.SemaphoreType.DMA((_NBUF,)),
            pltpu.SemaphoreType.DMA((_NBUF,)),
        ],
        compiler_params=pltpu.CompilerParams(
            use_tc_tiling_on_sc=False, needs_layout_passes=False),
    )
    def k(table_hbm, idx_hbm, out_hbm, table_v, idx_v, rows_v, sem_i, sem_o):
        wid = jax.lax.axis_index("s") * 2 + jax.lax.axis_index("c")
        p_base = wid * _PPW

        pltpu.sync_copy(table_hbm.at[pl.ds(0, _TROWS)], table_v)

        for b in range(_NBUF):
            pltpu.async_copy(
                idx_hbm.at[pl.ds((p_base + b * _CP) * N_ROWS, _CROWS)],
                idx_v.at[b], sem_i.at[b])

        iota = jax.lax.iota(jnp.int32, _L)

        def gather_group(b, g):
            rvec = idx_v[b, pl.ds(g * _L, _L)]
            rowv = g * _L + iota
            colv = iota
            v = plsc.load_gather(table_v, [rvec, colv])
            for c in range(D_REL - 1):
                nxt = colv ^ (c ^ (c + 1))
                vn = plsc.load_gather(table_v, [rvec, nxt])
                plsc.store_scatter(rows_v.at[b], [rowv, colv], v)
                colv = nxt
                v = vn
            plsc.store_scatter(rows_v.at[b], [rowv, colv], v)

        def gather_group_pair(b, g):
            rvec0 = idx_v[b, pl.ds(g * _L, _L)]
            rvec1 = idx_v[b, pl.ds((g + 1) * _L, _L)]
            rowv0 = g * _L + iota
            rowv1 = (g + 1) * _L + iota
            colv = iota
            v0 = plsc.load_gather(table_v, [rvec0, colv])
            v1 = plsc.load_gather(table_v, [rvec1, colv])
            for c in range(D_REL - 1):
                nxt = colv ^ (c ^ (c + 1))
                v0n = plsc.load_gather(table_v, [rvec0, nxt])
                plsc.store_scatter(rows_v.at[b], [rowv0, colv], v0)
                v1n = plsc.load_gather(table_v, [rvec1, nxt])
                plsc.store_scatter(rows_v.at[b], [rowv1, colv], v1)
                colv = nxt
                v0 = v0n
                v1 = v1n
            plsc.store_scatter(rows_v.at[b], [rowv0, colv], v0)
            plsc.store_scatter(rows_v.at[b], [rowv1, colv], v1)

        @pl.loop(0, _NCH, step=_NBUF)
        def _(ch0):
            for b in range(_NBUF):
                ch = ch0 + b
                p0 = p_base + ch * _CP
                pltpu.make_async_copy(
                    idx_hbm.at[pl.ds(0, _CROWS)], idx_v.at[b],
                    sem_i.at[b]).wait()

                @pl.when(ch >= _NBUF)
                def _():
                    for u in range(_CP):
                        pltpu.make_async_copy(
                            rows_v.at[b, pl.ds(0, N_ROWS)],
                            out_hbm.at[0], sem_o.at[b]).wait()

                @pl.loop(0, _CROWS // _L - 1, step=2)
                def _(g):
                    gather_group_pair(b, g)
                gather_group(b, _CROWS // _L - 1)

                for u in range(_CP):
                    pltpu.async_copy(
                        rows_v.at[b, pl.ds(u * N_ROWS, N_ROWS)],
                        out_hbm.at[p0 + u], sem_o.at[b])

                @pl.when(ch + _NBUF < _NCH)
                def _():
                    pltpu.async_copy(
                        idx_hbm.at[pl.ds((p0 + _NBUF * _CP) * N_ROWS,
                                         _CROWS)],
                        idx_v.at[b], sem_i.at[b])

        for b in range(_NBUF):
            for u in range(_CP):
                pltpu.make_async_copy(
                    rows_v.at[b, pl.ds(0, N_ROWS)],
                    out_hbm.at[0], sem_o.at[b]).wait()

    return k(table, idx_flat)


def kernel(price_timestamps, news_timestamps, month_table, weekday_table,
           hour_table, minute_table, session_table, relpos_table, W_proj,
           b_proj, modality_scale):
    rel_idx = pl.pallas_call(
        _relidx_body,
        out_shape=jax.ShapeDtypeStruct((P_ROWS, N_ROWS), jnp.int32),
    )(price_timestamps, news_timestamps.T)

    relpos = _sc_gather(relpos_table, rel_idx.reshape(NUM_IDX))

    bdiag = jnp.zeros((_COMB_PAD, 5 * D8), jnp.float32)
    bdiag = jax.lax.dynamic_update_slice(bdiag, month_table, (_OFF_M, 0))
    bdiag = jax.lax.dynamic_update_slice(bdiag, weekday_table, (_OFF_W, D8))
    bdiag = jax.lax.dynamic_update_slice(bdiag, hour_table, (_OFF_H, 2 * D8))
    bdiag = jax.lax.dynamic_update_slice(bdiag, minute_table, (_OFF_MIN, 3 * D8))
    bdiag = jax.lax.dynamic_update_slice(bdiag, session_table, (_OFF_S, 4 * D8))

    price_emb, news_emb = pl.pallas_call(
        _embed_body,
        out_shape=[
            jax.ShapeDtypeStruct((P_ROWS, D_MODEL), jnp.float32),
            jax.ShapeDtypeStruct((N_ROWS, D_MODEL), jnp.float32),
        ],
        in_specs=[
            pl.BlockSpec(memory_space=pltpu.VMEM),
            pl.BlockSpec(memory_space=pltpu.VMEM),
            pl.BlockSpec(memory_space=pltpu.VMEM),
            pl.BlockSpec(memory_space=pltpu.VMEM),
            pl.BlockSpec(memory_space=pltpu.VMEM),
            pl.BlockSpec(memory_space=pltpu.SMEM),
        ],
    )(price_timestamps, news_timestamps, bdiag, W_proj,
      b_proj.reshape(1, D_MODEL), modality_scale)

    return (price_emb, news_emb, relpos)


# speedup vs baseline: 1.2004x; 1.2004x over previous
import jax
import jax.numpy as jnp
from jax.experimental import pallas as pl
from jax.experimental.pallas import tpu as pltpu
from jax.experimental.pallas import tpu_sc as plsc

P_ROWS = 4096
N_ROWS = 200
D_MODEL = 256
D8 = D_MODEL // 8
D_REL = 64
NUM_IDX = P_ROWS * N_ROWS
GATHER_W = 128

_OFF_M, _OFF_W, _OFF_H, _OFF_MIN, _OFF_S = 0, 12, 17, 41, 101
_COMB = 105
_COMB_PAD = 128


def _relidx_body(pts_ref, nts_t_ref, out_ref):
    mb = pts_ref[:, 2:3] * 60 + pts_ref[:, 3:4]
    ma = nts_t_ref[2:3, :] * 60 + nts_t_ref[3:4, :]
    out_ref[...] = jnp.clip(mb - ma, -500, 500) + 500


def _session_col(hour, minute):
    t = hour * 60 + minute
    return jnp.where(t < 4 * 60, 0,
           jnp.where(t < 9 * 60 + 30, 1,
           jnp.where(t < 16 * 60, 2,
           jnp.where(t < 20 * 60, 3, 0))))


def _embed_body(pts_ref, nts_ref, bdiag_ref, w_ref, b_ref, scale_ref,
                pout_ref, nout_ref):
    bdiag = bdiag_ref[...]
    w = w_ref[...]
    bias = b_ref[...]

    def emb(ts, nrows, scale_val):
        cm = ts[:, 0:1] - 1 + _OFF_M
        cw = ts[:, 1:2] + _OFF_W
        ch = ts[:, 2:3] + _OFF_H
        cmin = ts[:, 3:4] + _OFF_MIN
        cs = _session_col(ts[:, 2:3], ts[:, 3:4]) + _OFF_S
        col = jax.lax.broadcasted_iota(jnp.int32, (nrows, _COMB_PAD), 1)
        h = ((col == cm).astype(jnp.float32)
             + (col == cw).astype(jnp.float32)
             + (col == ch).astype(jnp.float32)
             + (col == cmin).astype(jnp.float32)
             + (col == cs).astype(jnp.float32))
        feats = jnp.dot(h, bdiag, preferred_element_type=jnp.float32)
        out = jnp.dot(feats, w, preferred_element_type=jnp.float32)
        return (out + bias) * scale_val

    pout_ref[...] = emb(pts_ref[...], P_ROWS, scale_ref[1])
    nout_ref[...] = emb(nts_ref[...], N_ROWS, scale_ref[0])


_NW = 32
_PPW = P_ROWS // _NW
_CP = 2
_CROWS = _CP * N_ROWS
_NBUF = 2
_NCH = _PPW // _CP
_TROWS = 1001
_L = 16
_ROWW = N_ROWS * D_REL


def _sc_gather(table_flat, idx_flat):
    mesh = plsc.VectorSubcoreMesh(core_axis_name="c", subcore_axis_name="s")

    @pl.kernel(
        out_type=jax.ShapeDtypeStruct((P_ROWS, _ROWW), jnp.float32),
        mesh=mesh,
        scratch_types=[
            pltpu.VMEM((_TROWS * D_REL,), jnp.float32),
            pltpu.VMEM((_NBUF, _CROWS), jnp.int32),
            pltpu.VMEM((_NBUF, _CROWS * D_REL), jnp.float32),
            pltpu.SemaphoreType.DMA((_NBUF,)),
            pltpu.SemaphoreType.DMA((_NBUF,)),
        ],
        compiler_params=pltpu.CompilerParams(
            use_tc_tiling_on_sc=False, needs_layout_passes=False),
    )
    def k(table_hbm, idx_hbm, out_hbm, table_v, idx_v, rows_v, sem_i, sem_o):
        wid = jax.lax.axis_index("s") * 2 + jax.lax.axis_index("c")
        p_base = wid * _PPW

        pltpu.sync_copy(table_hbm.at[pl.ds(0, _TROWS * D_REL)], table_v)

        for b in range(_NBUF):
            pltpu.async_copy(
                idx_hbm.at[pl.ds((p_base + b * _CP) * N_ROWS, _CROWS)],
                idx_v.at[b], sem_i.at[b])

        iota = jax.lax.iota(jnp.int32, _L)

        @pl.loop(0, _NCH, step=_NBUF)
        def _(ch0):
            for b in range(_NBUF):
                ch = ch0 + b
                p0 = p_base + ch * _CP
                pltpu.make_async_copy(
                    idx_hbm.at[pl.ds(0, _CROWS)], idx_v.at[b],
                    sem_i.at[b]).wait()

                @pl.when(ch >= _NBUF)
                def _():
                    for _u in range(_CP):
                        pltpu.make_async_copy(
                            rows_v.at[b, pl.ds(0, _ROWW)],
                            out_hbm.at[0], sem_o.at[b]).wait()

                @pl.loop(0, _CROWS // _L - 1, step=2)
                def _(g):
                    rvec0 = idx_v[b, pl.ds(g * _L, _L)]
                    rvec1 = idx_v[b, pl.ds((g + 1) * _L, _L)]
                    lo0 = rvec0 * D_REL + iota
                    lo1 = rvec1 * D_REL + iota
                    so0 = (g * _L + iota) * D_REL + iota
                    so1 = ((g + 1) * _L + iota) * D_REL + iota
                    v0 = plsc.load_gather(table_v, [lo0])
                    v1 = plsc.load_gather(table_v, [lo1])
                    for c in range(D_REL - 1):
                        d = c ^ (c + 1)
                        lo0 = lo0 ^ d
                        lo1 = lo1 ^ d
                        v0n = plsc.load_gather(table_v, [lo0])
                        plsc.store_scatter(rows_v.at[b], [so0], v0)
                        so0 = so0 ^ d
                        v1n = plsc.load_gather(table_v, [lo1])
                        plsc.store_scatter(rows_v.at[b], [so1], v1)
                        so1 = so1 ^ d
                        v0 = v0n
                        v1 = v1n
                    plsc.store_scatter(rows_v.at[b], [so0], v0)
                    plsc.store_scatter(rows_v.at[b], [so1], v1)

                gt = _CROWS // _L - 1
                rvec = idx_v[b, pl.ds(gt * _L, _L)]
                lo = rvec * D_REL + iota
                so = (gt * _L + iota) * D_REL + iota
                v = plsc.load_gather(table_v, [lo])
                for c in range(D_REL - 1):
                    d = c ^ (c + 1)
                    lo = lo ^ d
                    vn = plsc.load_gather(table_v, [lo])
                    plsc.store_scatter(rows_v.at[b], [so], v)
                    so = so ^ d
                    v = vn
                plsc.store_scatter(rows_v.at[b], [so], v)

                for u in range(_CP):
                    pltpu.async_copy(
                        rows_v.at[b, pl.ds(u * _ROWW, _ROWW)],
                        out_hbm.at[p0 + u], sem_o.at[b])

                @pl.when(ch + _NBUF < _NCH)
                def _():
                    pltpu.async_copy(
                        idx_hbm.at[pl.ds((p0 + _NBUF * _CP) * N_ROWS,
                                         _CROWS)],
                        idx_v.at[b], sem_i.at[b])

        for b in range(_NBUF):
            for _u in range(_CP):
                pltpu.make_async_copy(
                    rows_v.at[b, pl.ds(0, _ROWW)],
                    out_hbm.at[0], sem_o.at[b]).wait()

    return k(table_flat, idx_flat)


def kernel(price_timestamps, news_timestamps, month_table, weekday_table,
           hour_table, minute_table, session_table, relpos_table, W_proj,
           b_proj, modality_scale):
    rel_idx = pl.pallas_call(
        _relidx_body,
        out_shape=jax.ShapeDtypeStruct((P_ROWS, N_ROWS), jnp.int32),
    )(price_timestamps, news_timestamps.T)

    gathered = _sc_gather(relpos_table.reshape(-1), rel_idx.reshape(NUM_IDX))
    relpos = gathered.reshape(P_ROWS, N_ROWS, D_REL)

    bdiag = jnp.zeros((_COMB_PAD, 5 * D8), jnp.float32)
    bdiag = jax.lax.dynamic_update_slice(bdiag, month_table, (_OFF_M, 0))
    bdiag = jax.lax.dynamic_update_slice(bdiag, weekday_table, (_OFF_W, D8))
    bdiag = jax.lax.dynamic_update_slice(bdiag, hour_table, (_OFF_H, 2 * D8))
    bdiag = jax.lax.dynamic_update_slice(bdiag, minute_table, (_OFF_MIN, 3 * D8))
    bdiag = jax.lax.dynamic_update_slice(bdiag, session_table, (_OFF_S, 4 * D8))

    price_emb, news_emb = pl.pallas_call(
        _embed_body,
        out_shape=[
            jax.ShapeDtypeStruct((P_ROWS, D_MODEL), jnp.float32),
            jax.ShapeDtypeStruct((N_ROWS, D_MODEL), jnp.float32),
        ],
        in_specs=[
            pl.BlockSpec(memory_space=pltpu.VMEM),
            pl.BlockSpec(memory_space=pltpu.VMEM),
            pl.BlockSpec(memory_space=pltpu.VMEM),
            pl.BlockSpec(memory_space=pltpu.VMEM),
            pl.BlockSpec(memory_space=pltpu.VMEM),
            pl.BlockSpec(memory_space=pltpu.SMEM),
        ],
    )(price_timestamps, news_timestamps, bdiag, W_proj,
      b_proj.reshape(1, D_MODEL), modality_scale)

    return (price_emb, news_emb, relpos)
